# Initial kernel scaffold; baseline (speedup 1.0000x reference)
import jax
import jax.numpy as jnp
from jax.experimental import pallas as pl


def kernel(x, y, values, lookup_table, hex_size, q_offset, r_offset, q_min, r_min, n_pixels):
    raise NotImplementedError("write your pallas kernel here")


# speedup vs baseline: 135.8165x; 135.8165x over previous
import functools

import jax
import jax.numpy as jnp
from jax import lax
from jax.experimental import pallas as pl
from jax.experimental.pallas import tpu as pltpu
from jax.experimental.pallas import tpu_sc as plsc

NC = 2
NS = 16
L = 16
NW = NC * NS

N_PHOTONS = 8388608
PER_W = N_PHOTONS // NW
CHUNK = 16384
NCHUNK = PER_W // CHUNK

N_PIXELS = 19
BINS_PAD = 24
HIST = BINS_PAD * L

RMAGIC = 12582912.0


@functools.lru_cache(maxsize=None)
def _sc_call(lut_rows, lut_cols):
    mesh = plsc.VectorSubcoreMesh(core_axis_name="c", subcore_axis_name="s")

    @functools.partial(
        pl.kernel,
        out_type=jax.ShapeDtypeStruct((NW, HIST), jnp.float32),
        mesh=mesh,
        compiler_params=pltpu.CompilerParams(needs_layout_passes=False),
        scratch_types=[
            pltpu.VMEM((CHUNK,), jnp.float32),
            pltpu.VMEM((CHUNK,), jnp.float32),
            pltpu.VMEM((CHUNK,), jnp.float32),
            pltpu.VMEM((CHUNK,), jnp.float32),
            pltpu.VMEM((CHUNK,), jnp.float32),
            pltpu.VMEM((CHUNK,), jnp.float32),
            pltpu.VMEM((32,), jnp.int32),
            pltpu.VMEM((8, L), jnp.float32),
            pltpu.VMEM((HIST,), jnp.float32),
            pltpu.SemaphoreType.DMA,
            pltpu.SemaphoreType.DMA,
        ],
    )
    def hexbin(x_hbm, y_hbm, v_hbm, tab_hbm, par_hbm, out_hbm,
               xb0, xb1, yb0, yb1, vb0, vb1, tab, par, hist, sem0, sem1):
        cid = lax.axis_index("c")
        sid = lax.axis_index("s")
        wid = sid * NC + cid
        base = wid * PER_W

        pltpu.sync_copy(tab_hbm, tab)
        pltpu.sync_copy(par_hbm, par)

        zero = jnp.zeros((L,), jnp.float32)
        for i in range(BINS_PAD):
            hist[pl.ds(i * L, L)] = zero

        a_v = par[0]
        b_v = par[1]
        c_v = par[2]
        d_v = par[3]
        e_v = par[4]
        qmin_v = par[5]
        rmin_v = par[6]
        rm = jnp.full((L,), RMAGIC, jnp.float32)
        lane = lax.iota(jnp.int32, L)
        sems = (sem0, sem1)
        bufs = ((xb0, yb0, vb0), (xb1, yb1, vb1))

        def start(g, slot):
            off = base + g * CHUNK
            xr, yr, vr = bufs[slot]
            return (
                pltpu.async_copy(x_hbm.at[pl.ds(off, CHUNK)], xr, sems[slot]),
                pltpu.async_copy(y_hbm.at[pl.ds(off, CHUNK)], yr, sems[slot]),
                pltpu.async_copy(v_hbm.at[pl.ds(off, CHUNK)], vr, sems[slot]),
            )

        def compute(slot):
            xr, yr, vr = bufs[slot]

            def body(i, carry):
                sl = pl.ds(i * L, L)
                xv = xr[sl]
                yv = yr[sl]
                vv = vr[sl]
                q = xv * a_v + yv * b_v + d_v
                r = yv * c_v + e_v
                s = -q - r
                qr = (q + rm) - rm
                rr = (r + rm) - rm
                sr = (s + rm) - rm
                qd = jnp.abs(qr - q)
                rd = jnp.abs(rr - r)
                sd = jnp.abs(sr - s)
                qr2 = jnp.where((qd > rd) & (qd > sd), -rr - sr, qr)
                rr2 = jnp.where((rd > qd) & (rd > sd), -qr - sr, rr)
                qi = (qr2 - qmin_v).astype(jnp.int32)
                ri = (rr2 - rmin_v).astype(jnp.int32)
                inb = (qi >= 0) & (qi < lut_rows) & (ri >= 0) & (ri < lut_cols)
                flat = jnp.clip(qi * lut_cols + ri, 0, lut_rows * lut_cols - 1)
                pix = plsc.load_gather(tab, [flat])
                valid = inb & (pix >= 0)
                addr = jnp.maximum(pix, 0) * L + lane
                plsc.addupdate_scatter(hist, [addr], vv, mask=valid)
                return carry

            lax.fori_loop(0, CHUNK // L, body, 0)

        pending = start(0, 0)
        for g in range(NCHUNK):
            slot = g & 1
            nxt = start(g + 1, 1 - slot) if g + 1 < NCHUNK else None
            for d in pending:
                d.wait()
            compute(slot)
            pending = nxt

        pltpu.sync_copy(hist, out_hbm.at[wid])

    return hexbin


def kernel(x, y, values, lookup_table, hex_size, q_offset, r_offset,
           q_min, r_min, n_pixels):
    lut_rows, lut_cols = lookup_table.shape
    h = jnp.float32(hex_size)
    s3 = jnp.sqrt(jnp.float32(3.0))
    par = jnp.stack([
        s3 / (3.0 * h),
        -1.0 / (3.0 * h),
        2.0 / (3.0 * h),
        -jnp.float32(q_offset),
        -jnp.float32(r_offset),
        jnp.float32(q_min),
        jnp.float32(r_min),
        jnp.float32(0.0),
    ])
    par = jnp.broadcast_to(par[:, None], (8, L)).astype(jnp.float32)
    flat_lut = lookup_table.astype(jnp.int32).reshape(-1)
    tab = jnp.concatenate(
        [flat_lut, jnp.full((32 - lut_rows * lut_cols,), -1, jnp.int32)])
    partials = _sc_call(lut_rows, lut_cols)(x, y, values, tab, par)
    return partials.reshape(NW, BINS_PAD, L).sum(axis=(0, 2))[:N_PIXELS]


# speedup vs baseline: 152.7772x; 1.1249x over previous
import functools

import jax
import jax.numpy as jnp
from jax import lax
from jax.experimental import pallas as pl
from jax.experimental.pallas import tpu as pltpu
from jax.experimental.pallas import tpu_sc as plsc

NC = 2
NS = 16
L = 16
NW = NC * NS

N_PHOTONS = 8388608
PER_W = N_PHOTONS // NW
CHUNK = 16384
NCHUNK = PER_W // CHUNK
UNROLL = 8

N_PIXELS = 19
BINS_PAD = 24
HIST = BINS_PAD * L

RMAGIC = 12582912.0


@functools.lru_cache(maxsize=None)
def _sc_call(lut_rows, lut_cols):
    mesh = plsc.VectorSubcoreMesh(core_axis_name="c", subcore_axis_name="s")

    @functools.partial(
        pl.kernel,
        out_type=jax.ShapeDtypeStruct((NW, HIST), jnp.float32),
        mesh=mesh,
        compiler_params=pltpu.CompilerParams(needs_layout_passes=False),
        scratch_types=[
            pltpu.VMEM((CHUNK,), jnp.float32),
            pltpu.VMEM((CHUNK,), jnp.float32),
            pltpu.VMEM((CHUNK,), jnp.float32),
            pltpu.VMEM((CHUNK,), jnp.float32),
            pltpu.VMEM((CHUNK,), jnp.float32),
            pltpu.VMEM((CHUNK,), jnp.float32),
            pltpu.VMEM((32,), jnp.int32),
            pltpu.VMEM((8, L), jnp.float32),
            pltpu.VMEM((HIST,), jnp.float32),
            pltpu.SemaphoreType.DMA,
            pltpu.SemaphoreType.DMA,
        ],
    )
    def hexbin(x_hbm, y_hbm, v_hbm, tab_hbm, par_hbm, out_hbm,
               xb0, xb1, yb0, yb1, vb0, vb1, tab, par, hist, sem0, sem1):
        cid = lax.axis_index("c")
        sid = lax.axis_index("s")
        wid = sid * NC + cid
        base = wid * PER_W

        pltpu.sync_copy(tab_hbm, tab)
        pltpu.sync_copy(par_hbm, par)

        zero = jnp.zeros((L,), jnp.float32)
        for i in range(BINS_PAD):
            hist[pl.ds(i * L, L)] = zero

        a_v = par[0]
        b_v = par[1]
        c_v = par[2]
        d_v = par[3]
        e_v = par[4]
        qmin_v = par[5]
        rmin_v = par[6]
        rm = jnp.full((L,), RMAGIC, jnp.float32)
        lane = lax.iota(jnp.int32, L)
        sems = (sem0, sem1)
        bufs = ((xb0, yb0, vb0), (xb1, yb1, vb1))

        def start(g, slot):
            off = base + g * CHUNK
            xr, yr, vr = bufs[slot]
            return (
                pltpu.async_copy(x_hbm.at[pl.ds(off, CHUNK)], xr, sems[slot]),
                pltpu.async_copy(y_hbm.at[pl.ds(off, CHUNK)], yr, sems[slot]),
                pltpu.async_copy(v_hbm.at[pl.ds(off, CHUNK)], vr, sems[slot]),
            )

        nrows = jnp.uint32(lut_rows)
        ncols = jnp.uint32(lut_cols)

        def compute(slot):
            xr, yr, vr = bufs[slot]

            def body(i, carry):
                b0 = i * (L * UNROLL)
                for u in range(UNROLL):
                    sl = pl.ds(b0 + u * L, L)
                    xv = xr[sl]
                    yv = yr[sl]
                    vv = vr[sl]
                    q = xv * a_v + yv * b_v + d_v
                    r = yv * c_v + e_v
                    s = -(q + r)
                    qr = (q + rm) - rm
                    rr = (r + rm) - rm
                    sr = (s + rm) - rm
                    qd = jnp.abs(qr - q)
                    rd = jnp.abs(rr - r)
                    sd = jnp.abs(sr - s)
                    qr2 = jnp.where((qd > rd) & (qd > sd), -(rr + sr), qr)
                    rr2 = jnp.where((rd > qd) & (rd > sd), -(qr + sr), rr)
                    qi = (qr2 - qmin_v).astype(jnp.int32)
                    ri = (rr2 - rmin_v).astype(jnp.int32)
                    inb = (lax.bitcast_convert_type(qi, jnp.uint32) < nrows) & (
                        lax.bitcast_convert_type(ri, jnp.uint32) < ncols)
                    flat = qi * lut_cols + ri
                    pix = plsc.load_gather(tab, [flat], mask=inb)
                    valid = inb & (pix >= 0)
                    addr = jnp.left_shift(pix, 4) + lane
                    plsc.addupdate_scatter(hist, [addr], vv, mask=valid)
                return carry

            lax.fori_loop(0, CHUNK // (L * UNROLL), body, 0)

        pending = start(0, 0)
        for g in range(NCHUNK):
            slot = g & 1
            nxt = start(g + 1, 1 - slot) if g + 1 < NCHUNK else None
            for d in pending:
                d.wait()
            compute(slot)
            pending = nxt

        pltpu.sync_copy(hist, out_hbm.at[wid])

    return hexbin


def kernel(x, y, values, lookup_table, hex_size, q_offset, r_offset,
           q_min, r_min, n_pixels):
    lut_rows, lut_cols = lookup_table.shape
    h = jnp.float32(hex_size)
    s3 = jnp.sqrt(jnp.float32(3.0))
    par = jnp.stack([
        s3 / (3.0 * h),
        -1.0 / (3.0 * h),
        2.0 / (3.0 * h),
        -jnp.float32(q_offset),
        -jnp.float32(r_offset),
        jnp.float32(q_min),
        jnp.float32(r_min),
        jnp.float32(0.0),
    ])
    par = jnp.broadcast_to(par[:, None], (8, L)).astype(jnp.float32)
    flat_lut = lookup_table.astype(jnp.int32).reshape(-1)
    tab = jnp.concatenate(
        [flat_lut, jnp.full((32 - lut_rows * lut_cols,), -1, jnp.int32)])
    partials = _sc_call(lut_rows, lut_cols)(x, y, values, tab, par)
    return partials.reshape(NW, BINS_PAD, L).sum(axis=(0, 2))[:N_PIXELS]


# speedup vs baseline: 319.0248x; 2.0882x over previous
import functools

import jax
import jax.numpy as jnp
from jax import lax
from jax.experimental import pallas as pl
from jax.experimental.pallas import tpu as pltpu
from jax.experimental.pallas import tpu_sc as plsc

NC = 2
NS = 16
L = 16
NW = NC * NS

N_PHOTONS = 8388608
PER_W = N_PHOTONS // NW
CHUNK = 16384
NCHUNK = PER_W // CHUNK
UNROLL = 8

N_PIXELS = 19
BINS_PAD = 24
HIST = BINS_PAD * L

RMAGIC = 12582912.0


@functools.lru_cache(maxsize=None)
def _sc_call(lut_rows, lut_cols):
    mesh = plsc.VectorSubcoreMesh(core_axis_name="c", subcore_axis_name="s")

    @functools.partial(
        pl.kernel,
        out_type=jax.ShapeDtypeStruct((NW, HIST), jnp.float32),
        mesh=mesh,
        compiler_params=pltpu.CompilerParams(needs_layout_passes=False),
        scratch_types=[
            pltpu.VMEM((CHUNK,), jnp.float32),
            pltpu.VMEM((CHUNK,), jnp.float32),
            pltpu.VMEM((CHUNK,), jnp.float32),
            pltpu.VMEM((CHUNK,), jnp.float32),
            pltpu.VMEM((CHUNK,), jnp.float32),
            pltpu.VMEM((CHUNK,), jnp.float32),
            pltpu.VMEM((32,), jnp.int32),
            pltpu.VMEM((8, L), jnp.float32),
            pltpu.VMEM((HIST,), jnp.float32),
            pltpu.SemaphoreType.DMA,
            pltpu.SemaphoreType.DMA,
        ],
    )
    def hexbin(x_hbm, y_hbm, v_hbm, tab_hbm, par_hbm, out_hbm,
               xb0, xb1, yb0, yb1, vb0, vb1, tab, par, hist, sem0, sem1):
        cid = lax.axis_index("c")
        sid = lax.axis_index("s")
        wid = sid * NC + cid
        base = wid * PER_W

        pltpu.sync_copy(tab_hbm, tab)
        pltpu.sync_copy(par_hbm, par)

        zero = jnp.zeros((L,), jnp.float32)
        for i in range(BINS_PAD):
            hist[pl.ds(i * L, L)] = zero

        a_v = par[0]
        b_v = par[1]
        c_v = par[2]
        d_v = par[3]
        e_v = par[4]
        qmin_v = par[5]
        rmin_v = par[6]
        rm = jnp.full((L,), RMAGIC, jnp.float32)
        lane = lax.iota(jnp.int32, L)
        sems = (sem0, sem1)
        bufs = ((xb0, yb0, vb0), (xb1, yb1, vb1))

        def start(g, slot):
            off = base + g * CHUNK
            xr, yr, vr = bufs[slot]
            return (
                pltpu.async_copy(x_hbm.at[pl.ds(off, CHUNK)], xr, sems[slot]),
                pltpu.async_copy(y_hbm.at[pl.ds(off, CHUNK)], yr, sems[slot]),
                pltpu.async_copy(v_hbm.at[pl.ds(off, CHUNK)], vr, sems[slot]),
            )

        nrows = jnp.uint32(lut_rows)
        ncols = jnp.uint32(lut_cols)

        def compute(slot):
            xr, yr, vr = bufs[slot]

            def body(i, carry):
                b0 = i * (L * UNROLL)
                flats, inbs, pixs = [], [], []
                for u in range(UNROLL):
                    sl = pl.ds(b0 + u * L, L)
                    xv = xr[sl]
                    yv = yr[sl]
                    q = xv * a_v + yv * b_v + d_v
                    r = yv * c_v + e_v
                    s = -(q + r)
                    qr = (q + rm) - rm
                    rr = (r + rm) - rm
                    sr = (s + rm) - rm
                    qd = jnp.abs(qr - q)
                    rd = jnp.abs(rr - r)
                    sd = jnp.abs(sr - s)
                    qr2 = jnp.where((qd > rd) & (qd > sd), -(rr + sr), qr)
                    rr2 = jnp.where((rd > qd) & (rd > sd), -(qr + sr), rr)
                    qi = (qr2 - qmin_v).astype(jnp.int32)
                    ri = (rr2 - rmin_v).astype(jnp.int32)
                    inb = (lax.bitcast_convert_type(qi, jnp.uint32) < nrows) & (
                        lax.bitcast_convert_type(ri, jnp.uint32) < ncols)
                    flats.append(qi * lut_cols + ri)
                    inbs.append(inb)
                for u in range(UNROLL):
                    pixs.append(plsc.load_gather(tab, [flats[u]], mask=inbs[u]))
                for u in range(UNROLL):
                    pix = pixs[u]
                    vv = vr[pl.ds(b0 + u * L, L)]
                    valid = inbs[u] & (pix >= 0)
                    addr = jnp.left_shift(pix, 4) + lane
                    plsc.addupdate_scatter(hist, [addr], vv, mask=valid)
                return carry

            lax.fori_loop(0, CHUNK // (L * UNROLL), body, 0)

        pending = start(0, 0)
        for g in range(NCHUNK):
            slot = g & 1
            nxt = start(g + 1, 1 - slot) if g + 1 < NCHUNK else None
            for d in pending:
                d.wait()
            compute(slot)
            pending = nxt

        pltpu.sync_copy(hist, out_hbm.at[wid])

    return hexbin


def kernel(x, y, values, lookup_table, hex_size, q_offset, r_offset,
           q_min, r_min, n_pixels):
    lut_rows, lut_cols = lookup_table.shape
    h = jnp.float32(hex_size)
    s3 = jnp.sqrt(jnp.float32(3.0))
    par = jnp.stack([
        s3 / (3.0 * h),
        -1.0 / (3.0 * h),
        2.0 / (3.0 * h),
        -jnp.float32(q_offset),
        -jnp.float32(r_offset),
        jnp.float32(q_min),
        jnp.float32(r_min),
        jnp.float32(0.0),
    ])
    par = jnp.broadcast_to(par[:, None], (8, L)).astype(jnp.float32)
    flat_lut = lookup_table.astype(jnp.int32).reshape(-1)
    tab = jnp.concatenate(
        [flat_lut, jnp.full((32 - lut_rows * lut_cols,), -1, jnp.int32)])
    partials = _sc_call(lut_rows, lut_cols)(x, y, values, tab, par)
    return partials.reshape(NW, BINS_PAD, L).sum(axis=(0, 2))[:N_PIXELS]


# speedup vs baseline: 339.7925x; 1.0651x over previous
import functools

import jax
import jax.numpy as jnp
from jax import lax
from jax.experimental import pallas as pl
from jax.experimental.pallas import tpu as pltpu
from jax.experimental.pallas import tpu_sc as plsc

NC = 2
NS = 16
L = 16
NW = NC * NS

N_PHOTONS = 8388608
PER_W = N_PHOTONS // NW
CHUNK = 16384
NCHUNK = PER_W // CHUNK
UNROLL = 8

N_PIXELS = 19
BINS_PAD = 24
HIST = BINS_PAD * L

RMAGIC = 12582912.0


@functools.lru_cache(maxsize=None)
def _sc_call(lut_rows, lut_cols):
    mesh = plsc.VectorSubcoreMesh(core_axis_name="c", subcore_axis_name="s")

    @functools.partial(
        pl.kernel,
        out_type=jax.ShapeDtypeStruct((NW, HIST), jnp.float32),
        mesh=mesh,
        compiler_params=pltpu.CompilerParams(needs_layout_passes=False),
        scratch_types=[
            pltpu.VMEM((CHUNK,), jnp.float32),
            pltpu.VMEM((CHUNK,), jnp.float32),
            pltpu.VMEM((CHUNK,), jnp.float32),
            pltpu.VMEM((CHUNK,), jnp.float32),
            pltpu.VMEM((CHUNK,), jnp.float32),
            pltpu.VMEM((CHUNK,), jnp.float32),
            pltpu.VMEM((64,), jnp.int32),
            pltpu.VMEM((16, L), jnp.float32),
            pltpu.VMEM((HIST,), jnp.float32),
            pltpu.SemaphoreType.DMA,
            pltpu.SemaphoreType.DMA,
        ],
    )
    def hexbin(x_hbm, y_hbm, v_hbm, tab_hbm, par_hbm, out_hbm,
               xb0, xb1, yb0, yb1, vb0, vb1, tab, par, hist, sem0, sem1):
        cid = lax.axis_index("c")
        sid = lax.axis_index("s")
        wid = sid * NC + cid
        base = wid * PER_W

        pltpu.sync_copy(tab_hbm, tab)
        pltpu.sync_copy(par_hbm, par)

        zero = jnp.zeros((L,), jnp.float32)
        for i in range(BINS_PAD):
            hist[pl.ds(i * L, L)] = zero

        a_v = par[0]
        b_v = par[1]
        c_v = par[2]
        d_v = par[3]
        e_v = par[4]
        qlo_v = par[5]
        qhi_v = par[6]
        rlo_v = par[7]
        rhi_v = par[8]
        k2_v = par[9]
        rm = jnp.full((L,), RMAGIC, jnp.float32)
        lane = lax.iota(jnp.int32, L)
        sems = (sem0, sem1)
        bufs = ((xb0, yb0, vb0), (xb1, yb1, vb1))

        def start(g, slot):
            off = base + g * CHUNK
            xr, yr, vr = bufs[slot]
            return (
                pltpu.async_copy(x_hbm.at[pl.ds(off, CHUNK)], xr, sems[slot]),
                pltpu.async_copy(y_hbm.at[pl.ds(off, CHUNK)], yr, sems[slot]),
                pltpu.async_copy(v_hbm.at[pl.ds(off, CHUNK)], vr, sems[slot]),
            )

        wcols = float(lut_cols + 2)

        def compute(slot):
            xr, yr, vr = bufs[slot]

            def body(i, carry):
                b0 = i * (L * UNROLL)
                flats, pixs = [], []
                for u in range(UNROLL):
                    sl = pl.ds(b0 + u * L, L)
                    xv = xr[sl]
                    yv = yr[sl]
                    q = xv * a_v + yv * b_v + d_v
                    r = yv * c_v + e_v
                    t = q + r
                    qr = (q + rm) - rm
                    rr = (r + rm) - rm
                    tr = (t + rm) - rm
                    qd = jnp.abs(qr - q)
                    rd = jnp.abs(rr - r)
                    sd = jnp.abs(t - tr)
                    qr2 = jnp.where((qd > rd) & (qd > sd), tr - rr, qr)
                    rr2 = jnp.where((rd > qd) & (rd > sd), tr - qr, rr)
                    qc = jnp.minimum(jnp.maximum(qr2, qlo_v), qhi_v)
                    rc = jnp.minimum(jnp.maximum(rr2, rlo_v), rhi_v)
                    flats.append((qc * wcols + rc - k2_v).astype(jnp.int32))
                for u in range(UNROLL):
                    pixs.append(plsc.load_gather(tab, [flats[u]]))
                for u in range(UNROLL):
                    pix = pixs[u]
                    vv = vr[pl.ds(b0 + u * L, L)]
                    valid = pix >= 0
                    addr = jnp.left_shift(pix, 4) + lane
                    plsc.addupdate_scatter(hist, [addr], vv, mask=valid)
                return carry

            lax.fori_loop(0, CHUNK // (L * UNROLL), body, 0)

        pending = start(0, 0)
        for g in range(NCHUNK):
            slot = g & 1
            nxt = start(g + 1, 1 - slot) if g + 1 < NCHUNK else None
            for d in pending:
                d.wait()
            compute(slot)
            pending = nxt

        pltpu.sync_copy(hist, out_hbm.at[wid])

    return hexbin


def kernel(x, y, values, lookup_table, hex_size, q_offset, r_offset,
           q_min, r_min, n_pixels):
    lut_rows, lut_cols = lookup_table.shape
    h = jnp.float32(hex_size)
    s3 = jnp.sqrt(jnp.float32(3.0))
    qminf = jnp.float32(q_min)
    rminf = jnp.float32(r_min)
    wcols = float(lut_cols + 2)
    par = jnp.stack([
        s3 / (3.0 * h),
        -1.0 / (3.0 * h),
        2.0 / (3.0 * h),
        -jnp.float32(q_offset),
        -jnp.float32(r_offset),
        qminf - 1.0,
        qminf + lut_rows,
        rminf - 1.0,
        rminf + lut_cols,
        (qminf - 1.0) * wcols + (rminf - 1.0),
        *([jnp.float32(0.0)] * 6),
    ])
    par = jnp.broadcast_to(par[:, None], (16, L)).astype(jnp.float32)
    bordered = jnp.pad(lookup_table.astype(jnp.int32), 1, constant_values=-1)
    flat_lut = bordered.reshape(-1)
    tab = jnp.concatenate(
        [flat_lut, jnp.full((64 - flat_lut.shape[0],), -1, jnp.int32)])
    partials = _sc_call(lut_rows, lut_cols)(x, y, values, tab, par)
    return partials.reshape(NW, BINS_PAD, L).sum(axis=(0, 2))[:N_PIXELS]


# speedup vs baseline: 357.2832x; 1.0515x over previous
import functools

import jax
import jax.numpy as jnp
from jax import lax
from jax.experimental import pallas as pl
from jax.experimental.pallas import tpu as pltpu
from jax.experimental.pallas import tpu_sc as plsc

NC = 2
NS = 16
L = 16
NW = NC * NS

N_PHOTONS = 8388608
PER_W = N_PHOTONS // NW
CHUNK = 16384
NCHUNK = PER_W // CHUNK
UNROLL = 16

N_PIXELS = 19
BINS_PAD = 24
HIST = BINS_PAD * L

RMAGIC = 12582912.0


@functools.lru_cache(maxsize=None)
def _sc_call(lut_rows, lut_cols):
    mesh = plsc.VectorSubcoreMesh(core_axis_name="c", subcore_axis_name="s")

    @functools.partial(
        pl.kernel,
        out_type=jax.ShapeDtypeStruct((NW, HIST), jnp.float32),
        mesh=mesh,
        compiler_params=pltpu.CompilerParams(needs_layout_passes=False),
        scratch_types=[
            pltpu.VMEM((CHUNK,), jnp.float32),
            pltpu.VMEM((CHUNK,), jnp.float32),
            pltpu.VMEM((CHUNK,), jnp.float32),
            pltpu.VMEM((CHUNK,), jnp.float32),
            pltpu.VMEM((CHUNK,), jnp.float32),
            pltpu.VMEM((CHUNK,), jnp.float32),
            pltpu.VMEM((64,), jnp.int32),
            pltpu.VMEM((16, L), jnp.float32),
            pltpu.VMEM((HIST,), jnp.float32),
            pltpu.SemaphoreType.DMA,
            pltpu.SemaphoreType.DMA,
        ],
    )
    def hexbin(x_hbm, y_hbm, v_hbm, tab_hbm, par_hbm, out_hbm,
               xb0, xb1, yb0, yb1, vb0, vb1, tab, par, hist, sem0, sem1):
        cid = lax.axis_index("c")
        sid = lax.axis_index("s")
        wid = sid * NC + cid
        base = wid * PER_W

        pltpu.sync_copy(tab_hbm, tab)
        pltpu.sync_copy(par_hbm, par)

        zero = jnp.zeros((L,), jnp.float32)
        for i in range(BINS_PAD):
            hist[pl.ds(i * L, L)] = zero

        a_v = par[0]
        b_v = par[1]
        c_v = par[2]
        d_v = par[3]
        e_v = par[4]
        qlo_v = par[5]
        qhi_v = par[6]
        rlo_v = par[7]
        rhi_v = par[8]
        k2_v = par[9]
        rm = jnp.full((L,), RMAGIC, jnp.float32)
        lane = lax.iota(jnp.int32, L)
        sems = (sem0, sem1)
        bufs = ((xb0, yb0, vb0), (xb1, yb1, vb1))

        def start(g, slot):
            off = base + g * CHUNK
            xr, yr, vr = bufs[slot]
            return (
                pltpu.async_copy(x_hbm.at[pl.ds(off, CHUNK)], xr, sems[slot]),
                pltpu.async_copy(y_hbm.at[pl.ds(off, CHUNK)], yr, sems[slot]),
                pltpu.async_copy(v_hbm.at[pl.ds(off, CHUNK)], vr, sems[slot]),
            )

        wcols = float(lut_cols + 2)

        def compute(slot):
            xr, yr, vr = bufs[slot]

            def body(i, carry):
                b0 = i * (L * UNROLL)
                pixs = []
                for u in range(UNROLL):
                    sl = pl.ds(b0 + u * L, L)
                    xv = xr[sl]
                    yv = yr[sl]
                    q = xv * a_v + yv * b_v + d_v
                    r = yv * c_v + e_v
                    t = q + r
                    qr = (q + rm) - rm
                    rr = (r + rm) - rm
                    tr = (t + rm) - rm
                    qd = jnp.abs(qr - q)
                    rd = jnp.abs(rr - r)
                    sd = jnp.abs(t - tr)
                    qr2 = jnp.where((qd > rd) & (qd > sd), tr - rr, qr)
                    rr2 = jnp.where((rd > qd) & (rd > sd), tr - qr, rr)
                    qc = jnp.minimum(jnp.maximum(qr2, qlo_v), qhi_v)
                    rc = jnp.minimum(jnp.maximum(rr2, rlo_v), rhi_v)
                    flat = (qc * wcols + rc - k2_v).astype(jnp.int32)
                    pixs.append(plsc.load_gather(tab, [flat]))
                for u in range(UNROLL):
                    pix = pixs[u]
                    vv = vr[pl.ds(b0 + u * L, L)]
                    valid = pix >= 0
                    addr = jnp.left_shift(pix, 4) + lane
                    plsc.addupdate_scatter(hist, [addr], vv, mask=valid)
                return carry

            lax.fori_loop(0, CHUNK // (L * UNROLL), body, 0)

        pending = start(0, 0)
        for g in range(NCHUNK):
            slot = g & 1
            nxt = start(g + 1, 1 - slot) if g + 1 < NCHUNK else None
            for d in pending:
                d.wait()
            compute(slot)
            pending = nxt

        pltpu.sync_copy(hist, out_hbm.at[wid])

    return hexbin


def kernel(x, y, values, lookup_table, hex_size, q_offset, r_offset,
           q_min, r_min, n_pixels):
    lut_rows, lut_cols = lookup_table.shape
    h = jnp.float32(hex_size)
    s3 = jnp.sqrt(jnp.float32(3.0))
    qminf = jnp.float32(q_min)
    rminf = jnp.float32(r_min)
    wcols = float(lut_cols + 2)
    par = jnp.stack([
        s3 / (3.0 * h),
        -1.0 / (3.0 * h),
        2.0 / (3.0 * h),
        -jnp.float32(q_offset),
        -jnp.float32(r_offset),
        qminf - 1.0,
        qminf + lut_rows,
        rminf - 1.0,
        rminf + lut_cols,
        (qminf - 1.0) * wcols + (rminf - 1.0),
        *([jnp.float32(0.0)] * 6),
    ])
    par = jnp.broadcast_to(par[:, None], (16, L)).astype(jnp.float32)
    bordered = jnp.pad(lookup_table.astype(jnp.int32), 1, constant_values=-1)
    flat_lut = bordered.reshape(-1)
    tab = jnp.concatenate(
        [flat_lut, jnp.full((64 - flat_lut.shape[0],), -1, jnp.int32)])
    partials = _sc_call(lut_rows, lut_cols)(x, y, values, tab, par)
    return partials.reshape(NW, BINS_PAD, L).sum(axis=(0, 2))[:N_PIXELS]
